# baseline (speedup 1.0000x reference)
import functools

import jax
import jax.numpy as jnp
from jax import lax
from jax.experimental import pallas as pl
from jax.experimental.pallas import tpu as pltpu
from jax.experimental.pallas import tpu_sc as plsc

N = 10000
E = 320000
D = 128

NC = 2
NS = 16
NW = NC * NS

EPW = E // NW
CHUNK = 80
NCHUNK = EPW // CHUNK
NBUF = 4
NSET = 8
UNROLL = 8
NITER = (NCHUNK - (NCHUNK % UNROLL)) // UNROLL
DRAIN = NCHUNK - NITER * UNROLL
RPT = 624
REM = N - NS * RPT


def _sc_aggregate_body(x_hbm, ei_hbm, adj_hbm, p_hbm, acc,
                       *scratch):
  srcb = list(scratch[0:NSET])
  dstb = list(scratch[NSET:2 * NSET])
  adjb = list(scratch[2 * NSET:3 * NSET])
  isem = list(scratch[3 * NSET:4 * NSET])
  rows = list(scratch[4 * NSET:4 * NSET + NBUF])
  gsem = list(scratch[4 * NSET + NBUF:4 * NSET + 2 * NBUF])
  ssem = list(scratch[4 * NSET + 2 * NBUF:4 * NSET + 3 * NBUF])
  r0 = rows[NBUF - 1]
  c = lax.axis_index("c")
  s = lax.axis_index("s")
  w_id = c * NS + s

  zero16 = jnp.zeros((16,), jnp.float32)

  def idx_start(j, k):
    base = pl.multiple_of(w_id * EPW + k * CHUNK, CHUNK)
    pltpu.async_copy(ei_hbm.at[pl.ds(base, CHUNK)], srcb[j], isem[j])
    pltpu.async_copy(ei_hbm.at[pl.ds(E + base, CHUNK)], dstb[j], isem[j])
    pltpu.async_copy(adj_hbm.at[pl.ds(base, CHUNK)], adjb[j], isem[j])

  def gather_start(b, j, k):
    base = pl.multiple_of(w_id * EPW + k * CHUNK, CHUNK)
    pltpu.make_async_copy(ei_hbm.at[pl.ds(base, CHUNK)], srcb[j],
                          isem[j]).wait()
    pltpu.make_async_copy(ei_hbm.at[pl.ds(E + base, CHUNK)], dstb[j],
                          isem[j]).wait()
    pltpu.make_async_copy(adj_hbm.at[pl.ds(base, CHUNK)], adjb[j],
                          isem[j]).wait()
    pltpu.async_copy(x_hbm.at[srcb[j]], rows[b], gsem[b])

  def process(b, j):
    pltpu.make_async_copy(x_hbm.at[srcb[j]], rows[b], gsem[b]).wait()

    def scale16(g, c2):
      av = adjb[j][pl.ds(g * 16, 16)]
      base = g * 16
      for j2 in range(16):
        a = jnp.full((16,), av[j2])
        r = base + j2
        for q in range(D // 16):
          rows[b][r, pl.ds(q * 16, 16)] = rows[b][r, pl.ds(q * 16, 16)] * a
      return c2
    lax.fori_loop(0, CHUNK // 16, scale16, 0)

    pltpu.async_copy(rows[b], acc.at[dstb[j]], ssem[b], add=True)

  def scatter_wait(b, j):
    pltpu.make_async_copy(rows[b], acc.at[dstb[j]], ssem[b]).wait()

  def position(k_dyn, kmod, first_dyn):
    j = kmod % NSET
    b = kmod % NBUF
    pb = (kmod - 1) % NBUF
    pj = (kmod - 1) % NSET
    j3 = (kmod + 3) % NSET
    nk4 = k_dyn + 4
    nk3 = k_dyn + 3

    def _do_idx():
      idx_start((kmod + 4) % NSET, nk4)

    def _do_refill():
      gather_start(pb, j3, nk3)

    static = isinstance(k_dyn, int)
    if not static and first_dyn:
      _do_idx()
      process(b, j)

      @pl.when(k_dyn > 0)
      def _dr():
        scatter_wait(pb, pj)
      _do_refill()
    elif not static:
      _do_idx()
      process(b, j)
      scatter_wait(pb, pj)
      _do_refill()
    else:
      if nk4 < NCHUNK:
        _do_idx()
      process(b, j)
      scatter_wait(pb, pj)
      if nk3 < NCHUNK:
        _do_refill()

  for k in range(NBUF):
    idx_start(k % NSET, k)

  def zrow(r, carry):
    for q in range(D // 16):
      r0[r, pl.ds(q * 16, 16)] = zero16
    return carry
  lax.fori_loop(0, CHUNK, zrow, 0)
  zsem = ssem[0]
  _nz = RPT // CHUNK
  for t in range(_nz):
    pltpu.async_copy(r0, acc.at[pl.ds(s * RPT + t * CHUNK, CHUNK)], zsem)
  _tail = RPT - _nz * CHUNK
  if _tail:
    pltpu.async_copy(r0.at[pl.ds(0, _tail)],
                     acc.at[pl.ds(s * RPT + RPT - _tail, _tail)], zsem)

  @pl.when(s == 0)
  def _zero_rem():
    pltpu.async_copy(r0.at[pl.ds(0, REM)], acc.at[pl.ds(NS * RPT, REM)],
                     zsem)

  for k in range(NBUF - 1):
    gather_start(k % NBUF, k % NSET, k)

  for t in range(_nz):
    pltpu.make_async_copy(r0, acc.at[pl.ds(s * RPT + t * CHUNK, CHUNK)],
                          zsem).wait()
  if _tail:
    pltpu.make_async_copy(r0.at[pl.ds(0, _tail)],
                          acc.at[pl.ds(s * RPT + RPT - _tail, _tail)],
                          zsem).wait()

  @pl.when(s == 0)
  def _wait_rem():
    pltpu.make_async_copy(r0.at[pl.ds(0, REM)],
                          acc.at[pl.ds(NS * RPT, REM)], zsem).wait()

  plsc.subcore_barrier()

  def pipe_body(i, carry):
    for m in range(UNROLL):
      position(i * UNROLL + m, m, m == 0)
    return carry
  lax.fori_loop(0, NITER, pipe_body, 0)

  for k in range(NITER * UNROLL, NCHUNK):
    position(k, k % UNROLL, False)

  scatter_wait((NCHUNK - 1) % NBUF, (NCHUNK - 1) % NSET)

  plsc.subcore_barrier()

  pltpu.sync_copy(acc.at[pl.ds(s * RPT, RPT)], p_hbm.at[c, pl.ds(s * RPT, RPT)])

  @pl.when(s == 0)
  def _write_rem():
    pltpu.sync_copy(acc.at[pl.ds(NS * RPT, REM)],
                    p_hbm.at[c, pl.ds(NS * RPT, REM)])


@jax.jit
def _sc_aggregate(x, ei, adj):
  mesh = plsc.VectorSubcoreMesh(core_axis_name="c", subcore_axis_name="s")
  return pl.kernel(
      _sc_aggregate_body,
      out_type=jax.ShapeDtypeStruct((NC, N, D), jnp.float32),
      mesh=mesh,
      scratch_types=[
          pltpu.VMEM_SHARED((N, D), jnp.float32),
          *[pltpu.VMEM((CHUNK,), jnp.int32) for _ in range(NSET)],
          *[pltpu.VMEM((CHUNK,), jnp.int32) for _ in range(NSET)],
          *[pltpu.VMEM((CHUNK,), jnp.float32) for _ in range(NSET)],
          *[pltpu.SemaphoreType.DMA for _ in range(NSET)],
          *[pltpu.VMEM((CHUNK, D), jnp.float32) for _ in range(NBUF)],
          *[pltpu.SemaphoreType.DMA for _ in range(NBUF)],
          *[pltpu.SemaphoreType.DMA for _ in range(NBUF)],
      ],
  )(x, ei, adj)


def _tc_combine_body(p_ref, w_ref, o_ref):
  a = p_ref[0] + p_ref[1]
  h = jnp.dot(a, w_ref[...], preferred_element_type=jnp.float32)
  o_ref[...] = jnp.maximum(h, 0.0)


@jax.jit
def _tc_combine(p, w):
  bn = N
  return pl.pallas_call(
      _tc_combine_body,
      grid=(N // bn,),
      in_specs=[
          pl.BlockSpec((NC, bn, D), lambda i: (0, i, 0)),
          pl.BlockSpec((D, D), lambda i: (0, 0)),
      ],
      out_specs=pl.BlockSpec((bn, D), lambda i: (i, 0)),
      out_shape=jax.ShapeDtypeStruct((N, D), jnp.float32),
  )(p, w)


def kernel(input, w, edge_index, adj_values):
  ei = edge_index.astype(jnp.int32).reshape(2 * E)
  p = _sc_aggregate(input, ei, adj_values)
  return _tc_combine(p, w)
